# Initial kernel scaffold; baseline (speedup 1.0000x reference)
import functools

import jax
import jax.numpy as jnp
from jax import lax
from jax.experimental import pallas as pl
from jax.experimental.pallas import tpu as pltpu
from jax.experimental.pallas import tpu_sc as plsc

N = 10000
E = 320000
D_EDGE = 4
MLP = 16
EPS = 1e-16

NC, NS = 2, 16
NW = NC * NS
ER = E // 128
RPT = ER // NW
REM = ER - RPT * NW

_SC_MESH = dict(core_axis_name="c", subcore_axis_name="s",
                num_cores=NC, num_subcores=NS)


def _k1_body_first(x_ref, w_ref, a1s_ref, a1d_ref, h_ref, s1_ref, d1_ref):
    h = jnp.dot(x_ref[...], w_ref[...], preferred_element_type=jnp.float32)
    h_ref[...] = h
    s1_ref[...] = jnp.dot(h, a1s_ref[...], preferred_element_type=jnp.float32)
    d1_ref[...] = jnp.dot(h, a1d_ref[...], preferred_element_type=jnp.float32)


def _k1_body_mid(acc_ref, den_ref, w_ref, a1s_ref, a1d_ref,
                 h_ref, s1_ref, d1_ref):
    xb = (acc_ref[0] + acc_ref[1]) / (den_ref[0] + den_ref[1] + EPS)
    h = jnp.dot(xb, w_ref[...], preferred_element_type=jnp.float32)
    h_ref[...] = h
    s1_ref[...] = jnp.dot(h, a1s_ref[...], preferred_element_type=jnp.float32)
    d1_ref[...] = jnp.dot(h, a1d_ref[...], preferred_element_type=jnp.float32)


def _node_matmuls(x_or_accden, w, a1s, a1d, first):
    bn = 2000
    grid = (N // bn,)
    dout = w.shape[1]
    wspec = [
        pl.BlockSpec(w.shape, lambda i: (0, 0)),
        pl.BlockSpec(a1s.shape, lambda i: (0, 0)),
        pl.BlockSpec(a1d.shape, lambda i: (0, 0)),
    ]
    out_shape = [
        jax.ShapeDtypeStruct((N, dout), jnp.float32),
        jax.ShapeDtypeStruct((N, MLP), jnp.float32),
        jax.ShapeDtypeStruct((N, MLP), jnp.float32),
    ]
    out_specs = [
        pl.BlockSpec((bn, dout), lambda i: (i, 0)),
        pl.BlockSpec((bn, MLP), lambda i: (i, 0)),
        pl.BlockSpec((bn, MLP), lambda i: (i, 0)),
    ]
    if first:
        x = x_or_accden
        return pl.pallas_call(
            _k1_body_first, grid=grid,
            in_specs=[pl.BlockSpec((bn, x.shape[1]), lambda i: (i, 0))] + wspec,
            out_specs=out_specs, out_shape=out_shape,
        )(x, w, a1s, a1d)
    acc, den = x_or_accden
    return pl.pallas_call(
        _k1_body_mid, grid=grid,
        in_specs=[pl.BlockSpec((2, bn, 64), lambda i: (0, i, 0)),
                  pl.BlockSpec((2, bn, 1), lambda i: (0, i, 0))] + wspec,
        out_specs=out_specs, out_shape=out_shape,
    )(acc, den, w, a1s, a1d)


def _k3_body(sg_ref, dg_ref, ea_ref, a1e_ref, b1_ref, a2_ref, b2_ref,
             a3_ref, b3_ref, lg_ref, k_ref):
    i = pl.program_id(0)
    z = sg_ref[...] + dg_ref[...]
    z = z + jnp.dot(ea_ref[...], a1e_ref[...],
                    preferred_element_type=jnp.float32)
    z = jnp.maximum(z + b1_ref[...], 0.0)
    z = jnp.maximum(jnp.dot(z, a2_ref[...],
                            preferred_element_type=jnp.float32) + b2_ref[...],
                    0.0)
    lg = jnp.dot(z, a3_ref[...], preferred_element_type=jnp.float32) \
        + b3_ref[...]
    lg = jnp.where(lg >= 0.0, lg, 0.2 * lg)
    lg_ref[...] = lg
    bmax = jnp.max(lg)

    @pl.when(i == 0)
    def _():
        k_ref[0, 0] = bmax

    @pl.when(i > 0)
    def _():
        k_ref[0, 0] = jnp.maximum(k_ref[0, 0], bmax)


def _edge_mlp(sg, dg, eav, a1e_bd, b1t, a2_bd, b2t, a3_bd, b3t):
    e8 = E // 8
    be = 4000
    grid = (e8 // be,)
    lg, kmax = pl.pallas_call(
        _k3_body, grid=grid,
        in_specs=[
            pl.BlockSpec((be, 128), lambda i: (i, 0)),
            pl.BlockSpec((be, 128), lambda i: (i, 0)),
            pl.BlockSpec((be, 32), lambda i: (i, 0)),
            pl.BlockSpec((32, 128), lambda i: (0, 0)),
            pl.BlockSpec((1, 128), lambda i: (0, 0)),
            pl.BlockSpec((128, 128), lambda i: (0, 0)),
            pl.BlockSpec((1, 128), lambda i: (0, 0)),
            pl.BlockSpec((128, 8), lambda i: (0, 0)),
            pl.BlockSpec((1, 8), lambda i: (0, 0)),
        ],
        out_specs=[pl.BlockSpec((be, 8), lambda i: (i, 0)),
                   pl.BlockSpec((1, 1), lambda i: (0, 0))],
        out_shape=[jax.ShapeDtypeStruct((e8, 8), jnp.float32),
                   jax.ShapeDtypeStruct((1, 1), jnp.float32)],
    )(sg, dg, eav, a1e_bd, b1t, a2_bd, b2t, a3_bd, b3t)
    return lg, kmax


def _k4_body(lg_ref, k_ref, ex_ref):
    ex_ref[...] = jnp.exp(lg_ref[...] - k_ref[0, 0])


def _exp_shift(lgv, kmax):
    rows = E // 512
    br = 125
    return pl.pallas_call(
        _k4_body, grid=(rows // br,),
        in_specs=[pl.BlockSpec((br, 512), lambda i: (i, 0)),
                  pl.BlockSpec((1, 1), lambda i: (0, 0))],
        out_specs=pl.BlockSpec((br, 512), lambda i: (i, 0)),
        out_shape=jax.ShapeDtypeStruct((rows, 512), jnp.float32),
    )(lgv, kmax)


def _epi_body(acc_ref, den_ref, out_ref):
    out_ref[...] = (acc_ref[0] + acc_ref[1]) / (den_ref[0] + den_ref[1] + EPS)


def _epilogue(acc, den):
    bn = 2000
    return pl.pallas_call(
        _epi_body, grid=(N // bn,),
        in_specs=[pl.BlockSpec((2, bn, 64), lambda i: (0, i, 0)),
                  pl.BlockSpec((2, bn, 1), lambda i: (0, i, 0))],
        out_specs=pl.BlockSpec((bn, 64), lambda i: (i, 0)),
        out_shape=jax.ShapeDtypeStruct((N, 64), jnp.float32),
    )(acc, den)


def _nrows(w):
    return jnp.where(w < REM, RPT + 1, RPT)


def _abs_row(w, j):
    return jnp.where(j == RPT, ER - REM + w, w * RPT + j)


def _k2_sc(src2d, dst2d, s1, d1):
    @functools.partial(
        pl.kernel,
        out_type=(jax.ShapeDtypeStruct((E, MLP), jnp.float32),
                  jax.ShapeDtypeStruct((E, MLP), jnp.float32)),
        mesh=plsc.VectorSubcoreMesh(**_SC_MESH),
        scratch_types=[
            pltpu.VMEM((RPT + 1, 128), jnp.int32),
            pltpu.VMEM((RPT + 1, 128), jnp.int32),
            pltpu.VMEM((128, MLP), jnp.float32),
            pltpu.VMEM((128, MLP), jnp.float32),
            pltpu.SemaphoreType.DMA,
            pltpu.SemaphoreType.DMA,
        ],
    )
    def body(src_hbm, dst_hbm, s1_hbm, d1_hbm, sg_hbm, dg_hbm,
             idx_s, idx_d, srow, drow, sem_a, sem_b):
        c = lax.axis_index("c")
        s = lax.axis_index("s")
        w = c * NS + s
        pltpu.sync_copy(src_hbm.at[pl.ds(w * RPT, RPT)], idx_s.at[pl.ds(0, RPT)])
        pltpu.sync_copy(dst_hbm.at[pl.ds(w * RPT, RPT)], idx_d.at[pl.ds(0, RPT)])

        @pl.when(w < REM)
        def _():
            pltpu.sync_copy(src_hbm.at[ER - REM + w], idx_s.at[RPT])
            pltpu.sync_copy(dst_hbm.at[ER - REM + w], idx_d.at[RPT])

        def row(j, carry):
            ar = _abs_row(w, j)
            cp1 = pltpu.async_copy(s1_hbm.at[idx_s.at[j]], srow, sem_a)
            cp2 = pltpu.async_copy(d1_hbm.at[idx_d.at[j]], drow, sem_b)
            cp1.wait()
            cp2.wait()
            pltpu.sync_copy(srow, sg_hbm.at[pl.ds(ar * 128, 128)])
            pltpu.sync_copy(drow, dg_hbm.at[pl.ds(ar * 128, 128)])
            return carry

        lax.fori_loop(0, _nrows(w), row, 0)

    return body(src2d, dst2d, s1, d1)


def _k5_sc(src2d, dst2d, h, ex2d):
    @functools.partial(
        pl.kernel,
        out_type=(jax.ShapeDtypeStruct((NC, N, 64), jnp.float32),
                  jax.ShapeDtypeStruct((NC, N), jnp.float32)),
        mesh=plsc.VectorSubcoreMesh(**_SC_MESH),
        scratch_types=[
            pltpu.VMEM_SHARED((N, 64), jnp.float32),
            pltpu.VMEM_SHARED((N,), jnp.float32),
            pltpu.VMEM((RPT + 1, 128), jnp.int32),
            pltpu.VMEM((RPT + 1, 128), jnp.int32),
            pltpu.VMEM((RPT + 1, 128), jnp.float32),
            pltpu.VMEM((128, 64), jnp.float32),
            pltpu.VMEM((128,), jnp.float32),
            pltpu.SemaphoreType.DMA,
        ],
    )
    def body(src_hbm, dst_hbm, h_hbm, ex_hbm, acc_out, den_out,
             acc_sh, den_sh, idx_s, idx_d, exb, rows, zb1, sem):
        c = lax.axis_index("c")
        s = lax.axis_index("s")
        w = c * NS + s

        def zrow(r, carry):
            for q in range(4):
                rows[r, pl.ds(q * 16, 16)] = jnp.zeros((16,), jnp.float32)
            return carry

        lax.fori_loop(0, 128, zrow, 0)
        for q in range(8):
            zb1[pl.ds(q * 16, 16)] = jnp.zeros((16,), jnp.float32)
        band = (s * 625) // 8 * 8
        for t in range(4):
            pltpu.sync_copy(rows, acc_sh.at[pl.ds(band + t * 128, 128)])
        pltpu.sync_copy(rows.at[pl.ds(0, 120)],
                        acc_sh.at[pl.ds(band + 512, 120)])
        for t in range(4):
            pltpu.sync_copy(zb1, den_sh.at[pl.ds(band + t * 128, 128)])
        pltpu.sync_copy(zb1.at[pl.ds(0, 120)],
                        den_sh.at[pl.ds(band + 512, 120)])
        plsc.subcore_barrier()

        pltpu.sync_copy(src_hbm.at[pl.ds(w * RPT, RPT)], idx_s.at[pl.ds(0, RPT)])
        pltpu.sync_copy(dst_hbm.at[pl.ds(w * RPT, RPT)], idx_d.at[pl.ds(0, RPT)])
        pltpu.sync_copy(ex_hbm.at[pl.ds(w * RPT, RPT)], exb.at[pl.ds(0, RPT)])

        @pl.when(w < REM)
        def _():
            pltpu.sync_copy(src_hbm.at[ER - REM + w], idx_s.at[RPT])
            pltpu.sync_copy(dst_hbm.at[ER - REM + w], idx_d.at[RPT])
            pltpu.sync_copy(ex_hbm.at[ER - REM + w], exb.at[RPT])

        def row(j, carry):
            pltpu.async_copy(h_hbm.at[idx_s.at[j]], rows, sem).wait()

            def scale(e, carry2):
                bex = plsc.load_gather(
                    exb, [jnp.full((16,), j, jnp.int32),
                          jnp.full((16,), e, jnp.int32)])
                for q in range(4):
                    rows[e, pl.ds(q * 16, 16)] = \
                        rows[e, pl.ds(q * 16, 16)] * bex
                return carry2

            lax.fori_loop(0, 128, scale, 0)
            pltpu.sync_copy(rows, acc_sh.at[idx_d.at[j]], add=True)
            pltpu.sync_copy(exb.at[j], den_sh.at[idx_d.at[j]], add=True)
            return carry

        lax.fori_loop(0, _nrows(w), row, 0)
        plsc.subcore_barrier()

        @pl.when(s == 0)
        def _():
            pltpu.sync_copy(acc_sh, acc_out.at[c])
            pltpu.sync_copy(den_sh, den_out.at[c])

    return body(src2d, dst2d, h, ex2d)


def _block_diag(m, k):
    din, dout = m.shape
    out = jnp.zeros((din * k, dout * k), jnp.float32)
    for i in range(k):
        out = out.at[i * din:(i + 1) * din, i * dout:(i + 1) * dout].set(m)
    return out


def kernel(x, edge_index, edge_attr, params):
    src2d = edge_index[0].astype(jnp.int32).reshape(ER, 128)
    dst2d = edge_index[1].astype(jnp.int32).reshape(ER, 128)
    eav = edge_attr.reshape(E // 8, 8 * D_EDGE)

    acc = den = None
    for li, p in enumerate(params):
        dout = p['W'].shape[1]
        a1 = p['A1']
        a1s, a1d, a1e = a1[:dout], a1[dout:2 * dout], a1[2 * dout:]
        a1e_bd = _block_diag(a1e, 8)
        b1t = jnp.tile(p['b1'], 8)[None, :]
        a2_bd = _block_diag(p['A2'], 8)
        b2t = jnp.tile(p['b2'], 8)[None, :]
        a3_bd = _block_diag(p['A3'], 8)
        b3t = jnp.tile(p['b3'], 8)[None, :]

        if li == 0:
            h, s1, d1 = _node_matmuls(x, p['W'], a1s, a1d, first=True)
        else:
            h, s1, d1 = _node_matmuls((acc, den), p['W'], a1s, a1d,
                                      first=False)

        sg, dg = _k2_sc(src2d, dst2d, s1, d1)
        lg, kmax = _edge_mlp(sg.reshape(E // 8, 128), dg.reshape(E // 8, 128),
                             eav, a1e_bd, b1t, a2_bd, b2t, a3_bd, b3t)
        ex = _exp_shift(lg.reshape(E // 512, 512), kmax)
        acc, den_flat = _k5_sc(src2d, dst2d, h, ex.reshape(ER, 128))
        den = den_flat.reshape(NC, N, 1)

    return _epilogue(acc, den)


# speedup vs baseline: 12.2685x; 12.2685x over previous
import functools

import jax
import jax.numpy as jnp
from jax import lax
from jax.experimental import pallas as pl
from jax.experimental.pallas import tpu as pltpu
from jax.experimental.pallas import tpu_sc as plsc

N = 10000
E = 320000
D_EDGE = 4
MLP = 16
EPS = 1e-16

NC, NS = 2, 16
NW = NC * NS
ERP = 2560
RPT = ERP // NW
EP = ERP * 128
EPAD = EP - E

_SC_MESH = dict(core_axis_name="c", subcore_axis_name="s",
                num_cores=NC, num_subcores=NS)


def _k1_body_first(x_ref, w_ref, a1s_ref, a1d_ref, h_ref, s1_ref, d1_ref):
    h = jnp.dot(x_ref[...], w_ref[...], preferred_element_type=jnp.float32)
    h_ref[...] = h
    s1_ref[...] = jnp.dot(h, a1s_ref[...], preferred_element_type=jnp.float32)
    d1_ref[...] = jnp.dot(h, a1d_ref[...], preferred_element_type=jnp.float32)


def _k1_body_mid(acc_ref, den_ref, w_ref, a1s_ref, a1d_ref,
                 h_ref, s1_ref, d1_ref):
    xb = (acc_ref[0] + acc_ref[1]) / (den_ref[0] + den_ref[1] + EPS)
    h = jnp.dot(xb, w_ref[...], preferred_element_type=jnp.float32)
    h_ref[...] = h
    s1_ref[...] = jnp.dot(h, a1s_ref[...], preferred_element_type=jnp.float32)
    d1_ref[...] = jnp.dot(h, a1d_ref[...], preferred_element_type=jnp.float32)


def _node_matmuls(x_or_accden, w, a1s, a1d, first):
    bn = 2000
    grid = (N // bn,)
    dout = w.shape[1]
    wspec = [
        pl.BlockSpec(w.shape, lambda i: (0, 0)),
        pl.BlockSpec(a1s.shape, lambda i: (0, 0)),
        pl.BlockSpec(a1d.shape, lambda i: (0, 0)),
    ]
    out_shape = [
        jax.ShapeDtypeStruct((N, dout), jnp.float32),
        jax.ShapeDtypeStruct((N, MLP), jnp.float32),
        jax.ShapeDtypeStruct((N, MLP), jnp.float32),
    ]
    out_specs = [
        pl.BlockSpec((bn, dout), lambda i: (i, 0)),
        pl.BlockSpec((bn, MLP), lambda i: (i, 0)),
        pl.BlockSpec((bn, MLP), lambda i: (i, 0)),
    ]
    if first:
        x = x_or_accden
        return pl.pallas_call(
            _k1_body_first, grid=grid,
            in_specs=[pl.BlockSpec((bn, x.shape[1]), lambda i: (i, 0))] + wspec,
            out_specs=out_specs, out_shape=out_shape,
        )(x, w, a1s, a1d)
    acc, den = x_or_accden
    return pl.pallas_call(
        _k1_body_mid, grid=grid,
        in_specs=[pl.BlockSpec((2, bn, 64), lambda i: (0, i, 0)),
                  pl.BlockSpec((2, bn, 1), lambda i: (0, i, 0))] + wspec,
        out_specs=out_specs, out_shape=out_shape,
    )(acc, den, w, a1s, a1d)


def _k3_body(sg_ref, dg_ref, ea_ref, a1e_ref, b1_ref, a2_ref, b2_ref,
             a3_ref, b3_ref, lg_ref, k_ref):
    i = pl.program_id(0)
    z = sg_ref[...] + dg_ref[...]
    z = z + jnp.dot(ea_ref[...], a1e_ref[...],
                    preferred_element_type=jnp.float32)
    z = jnp.maximum(z + b1_ref[...], 0.0)
    z = jnp.maximum(jnp.dot(z, a2_ref[...],
                            preferred_element_type=jnp.float32) + b2_ref[...],
                    0.0)
    lg = jnp.dot(z, a3_ref[...], preferred_element_type=jnp.float32) \
        + b3_ref[...]
    lg = jnp.where(lg >= 0.0, lg, 0.2 * lg)
    lg_ref[...] = lg
    bmax = jnp.full((1, 1), jnp.max(lg), jnp.float32)

    @pl.when(i == 0)
    def _():
        k_ref[...] = bmax

    @pl.when(i > 0)
    def _():
        k_ref[...] = jnp.maximum(k_ref[...], bmax)


def _edge_mlp(sg, dg, eav, a1e_bd, b1t, a2_bd, b2t, a3_bd, b3t):
    e8 = E // 8
    be = 4000
    grid = (e8 // be,)
    lg, kmax = pl.pallas_call(
        _k3_body, grid=grid,
        in_specs=[
            pl.BlockSpec((be, 128), lambda i: (i, 0)),
            pl.BlockSpec((be, 128), lambda i: (i, 0)),
            pl.BlockSpec((be, 32), lambda i: (i, 0)),
            pl.BlockSpec((32, 128), lambda i: (0, 0)),
            pl.BlockSpec((1, 128), lambda i: (0, 0)),
            pl.BlockSpec((128, 128), lambda i: (0, 0)),
            pl.BlockSpec((1, 128), lambda i: (0, 0)),
            pl.BlockSpec((128, 8), lambda i: (0, 0)),
            pl.BlockSpec((1, 8), lambda i: (0, 0)),
        ],
        out_specs=[pl.BlockSpec((be, 8), lambda i: (i, 0)),
                   pl.BlockSpec((1, 1), lambda i: (0, 0))],
        out_shape=[jax.ShapeDtypeStruct((e8, 8), jnp.float32),
                   jax.ShapeDtypeStruct((1, 1), jnp.float32)],
    )(sg, dg, eav, a1e_bd, b1t, a2_bd, b2t, a3_bd, b3t)
    return lg, kmax


def _k4_body(lg_ref, k_ref, ex_ref):
    ex_ref[...] = jnp.exp(lg_ref[...] - k_ref[...])


def _exp_shift(lgv, kmax):
    rows = E // 512
    return pl.pallas_call(
        _k4_body, grid=(1,),
        in_specs=[pl.BlockSpec((rows, 512), lambda i: (0, 0)),
                  pl.BlockSpec((1, 1), lambda i: (0, 0))],
        out_specs=pl.BlockSpec((rows, 512), lambda i: (0, 0)),
        out_shape=jax.ShapeDtypeStruct((rows, 512), jnp.float32),
    )(lgv, kmax)


def _epi_body(acc_ref, den_ref, out_ref):
    out_ref[...] = (acc_ref[0] + acc_ref[1]) / (den_ref[0] + den_ref[1] + EPS)


def _epilogue(acc, den):
    bn = 2000
    return pl.pallas_call(
        _epi_body, grid=(N // bn,),
        in_specs=[pl.BlockSpec((2, bn, 64), lambda i: (0, i, 0)),
                  pl.BlockSpec((2, bn, 1), lambda i: (0, i, 0))],
        out_specs=pl.BlockSpec((bn, 64), lambda i: (i, 0)),
        out_shape=jax.ShapeDtypeStruct((N, 64), jnp.float32),
    )(acc, den)


def _k2_sc(src2d, dst2d, s1, d1):
    @functools.partial(
        pl.kernel,
        out_type=(jax.ShapeDtypeStruct((EP, MLP), jnp.float32),
                  jax.ShapeDtypeStruct((EP, MLP), jnp.float32)),
        mesh=plsc.VectorSubcoreMesh(**_SC_MESH),
        compiler_params=pltpu.CompilerParams(use_tc_tiling_on_sc=False, needs_layout_passes=False),
        scratch_types=[
            pltpu.VMEM((RPT, 128), jnp.int32),
            pltpu.VMEM((RPT, 128), jnp.int32),
            pltpu.VMEM((128, MLP), jnp.float32),
            pltpu.VMEM((128, MLP), jnp.float32),
            pltpu.SemaphoreType.DMA,
            pltpu.SemaphoreType.DMA,
        ],
    )
    def body(src_hbm, dst_hbm, s1_hbm, d1_hbm, sg_hbm, dg_hbm,
             idx_s, idx_d, srow, drow, sem_a, sem_b):
        c = lax.axis_index("c")
        s = lax.axis_index("s")
        w = c * NS + s
        pltpu.sync_copy(src_hbm.at[pl.ds(w * RPT, RPT)], idx_s)
        pltpu.sync_copy(dst_hbm.at[pl.ds(w * RPT, RPT)], idx_d)

        def row(j, carry):
            ar = w * RPT + j
            cp1 = pltpu.async_copy(s1_hbm.at[idx_s.at[j]], srow, sem_a)
            cp2 = pltpu.async_copy(d1_hbm.at[idx_d.at[j]], drow, sem_b)
            cp1.wait()
            cp2.wait()
            pltpu.sync_copy(srow, sg_hbm.at[pl.ds(ar * 128, 128)])
            pltpu.sync_copy(drow, dg_hbm.at[pl.ds(ar * 128, 128)])
            return carry

        lax.fori_loop(0, RPT, row, 0)

    return body(src2d, dst2d, s1, d1)


def _k5_sc(src2d, dst2d, h, ex2d):
    @functools.partial(
        pl.kernel,
        out_type=(jax.ShapeDtypeStruct((NC, N, 64), jnp.float32),
                  jax.ShapeDtypeStruct((NC * N,), jnp.float32)),
        mesh=plsc.VectorSubcoreMesh(**_SC_MESH),
        compiler_params=pltpu.CompilerParams(use_tc_tiling_on_sc=False, needs_layout_passes=False),
        scratch_types=[
            pltpu.VMEM_SHARED((N, 64), jnp.float32),
            pltpu.VMEM_SHARED((N,), jnp.float32),
            pltpu.VMEM((RPT, 128), jnp.int32),
            pltpu.VMEM((RPT, 128), jnp.int32),
            pltpu.VMEM((RPT, 128), jnp.float32),
            pltpu.VMEM((128, 64), jnp.float32),
            pltpu.VMEM((128,), jnp.float32),
            pltpu.SemaphoreType.DMA,
        ],
    )
    def body(src_hbm, dst_hbm, h_hbm, ex_hbm, acc_out, den_out,
             acc_sh, den_sh, idx_s, idx_d, exb, rows, zb1, sem):
        c = lax.axis_index("c")
        s = lax.axis_index("s")
        w = c * NS + s

        def zrow(r, carry):
            for q in range(4):
                rows[r, pl.ds(q * 16, 16)] = jnp.zeros((16,), jnp.float32)
            return carry

        lax.fori_loop(0, 128, zrow, 0)
        for q in range(8):
            zb1[pl.ds(q * 16, 16)] = jnp.zeros((16,), jnp.float32)
        band = (s * 625) // 8 * 8
        for t in range(4):
            pltpu.sync_copy(rows, acc_sh.at[pl.ds(band + t * 128, 128)])
        pltpu.sync_copy(rows.at[pl.ds(0, 120)],
                        acc_sh.at[pl.ds(band + 512, 120)])
        for t in range(4):
            pltpu.sync_copy(zb1, den_sh.at[pl.ds(band + t * 128, 128)])
        pltpu.sync_copy(zb1.at[pl.ds(0, 120)],
                        den_sh.at[pl.ds(band + 512, 120)])
        plsc.subcore_barrier()

        pltpu.sync_copy(src_hbm.at[pl.ds(w * RPT, RPT)], idx_s)
        pltpu.sync_copy(dst_hbm.at[pl.ds(w * RPT, RPT)], idx_d)
        pltpu.sync_copy(ex_hbm.at[pl.ds(w * RPT, RPT)], exb)

        def row(j, carry):
            pltpu.async_copy(h_hbm.at[idx_s.at[j]], rows, sem).wait()

            def scale(e, carry2):
                bex = plsc.load_gather(
                    exb, [jnp.full((16,), j, jnp.int32),
                          jnp.full((16,), e, jnp.int32)])
                for q in range(4):
                    rows[e, pl.ds(q * 16, 16)] = \
                        rows[e, pl.ds(q * 16, 16)] * bex
                return carry2

            lax.fori_loop(0, 128, scale, 0)
            pltpu.sync_copy(rows, acc_sh.at[idx_d.at[j]], add=True)
            pltpu.sync_copy(exb.at[j], den_sh.at[idx_d.at[j]], add=True)
            return carry

        lax.fori_loop(0, RPT, row, 0)
        plsc.subcore_barrier()

        @pl.when(s == 0)
        def _():
            pltpu.sync_copy(acc_sh, acc_out.at[c])
            pltpu.sync_copy(den_sh, den_out.at[pl.ds(c * N, N)])

    return body(src2d, dst2d, h, ex2d)


def _block_diag(m, k):
    din, dout = m.shape
    out = jnp.zeros((din * k, dout * k), jnp.float32)
    for i in range(k):
        out = out.at[i * din:(i + 1) * din, i * dout:(i + 1) * dout].set(m)
    return out


def kernel(x, edge_index, edge_attr, params):
    zpad_i = jnp.zeros((EPAD,), jnp.int32)
    src2d = jnp.concatenate(
        [edge_index[0].astype(jnp.int32), zpad_i]).reshape(ERP, 128)
    dst2d = jnp.concatenate(
        [edge_index[1].astype(jnp.int32), zpad_i]).reshape(ERP, 128)
    eav = edge_attr.reshape(E // 8, 8 * D_EDGE)
    zpad_f = jnp.zeros((EPAD,), jnp.float32)

    acc = den = None
    for li, p in enumerate(params):
        dout = p['W'].shape[1]
        a1 = p['A1']
        a1s, a1d, a1e = a1[:dout], a1[dout:2 * dout], a1[2 * dout:]
        a1e_bd = _block_diag(a1e, 8)
        b1t = jnp.tile(p['b1'], 8)[None, :]
        a2_bd = _block_diag(p['A2'], 8)
        b2t = jnp.tile(p['b2'], 8)[None, :]
        a3_bd = _block_diag(p['A3'], 8)
        b3t = jnp.tile(p['b3'], 8)[None, :]

        if li == 0:
            h, s1, d1 = _node_matmuls(x, p['W'], a1s, a1d, first=True)
        else:
            h, s1, d1 = _node_matmuls((acc, den), p['W'], a1s, a1d,
                                      first=False)

        sg, dg = _k2_sc(src2d, dst2d, s1, d1)
        lg, kmax = _edge_mlp(sg.reshape(EP // 8, 128), dg.reshape(EP // 8, 128),
                             eav, a1e_bd, b1t, a2_bd, b2t, a3_bd, b3t)
        ex = _exp_shift(lg.reshape(E // 512, 512), kmax)
        expad = jnp.concatenate([ex.reshape(E), zpad_f]).reshape(ERP, 128)
        acc, den_flat = _k5_sc(src2d, dst2d, h, expad)
        den = den_flat.reshape(NC, N, 1)

    return _epilogue(acc, den)


# speedup vs baseline: 18.4647x; 1.5051x over previous
import functools

import jax
import jax.numpy as jnp
from jax import lax
from jax.experimental import pallas as pl
from jax.experimental.pallas import tpu as pltpu
from jax.experimental.pallas import tpu_sc as plsc

N = 10000
E = 320000
D_EDGE = 4
MLP = 16
EPS = 1e-16

NC, NS = 2, 16
NW = NC * NS
ER = E // 128
ERP = 2560
RPT = ERP // NW
EP = ERP * 128
EPAD = EP - E

_SC_MESH = dict(core_axis_name="c", subcore_axis_name="s",
                num_cores=NC, num_subcores=NS)


def _k1_body_first(x_ref, w_ref, a1s_ref, a1d_ref, h_ref, s1_ref, d1_ref):
    h = jnp.dot(x_ref[...], w_ref[...], preferred_element_type=jnp.float32)
    h_ref[...] = h
    s1_ref[...] = jnp.dot(h, a1s_ref[...], preferred_element_type=jnp.float32)
    d1_ref[...] = jnp.dot(h, a1d_ref[...], preferred_element_type=jnp.float32)


def _k1_body_mid(acc_ref, den_ref, w_ref, a1s_ref, a1d_ref,
                 h_ref, s1_ref, d1_ref):
    xb = (acc_ref[0] + acc_ref[1]) / (den_ref[0] + den_ref[1] + EPS)
    h = jnp.dot(xb, w_ref[...], preferred_element_type=jnp.float32)
    h_ref[...] = h
    s1_ref[...] = jnp.dot(h, a1s_ref[...], preferred_element_type=jnp.float32)
    d1_ref[...] = jnp.dot(h, a1d_ref[...], preferred_element_type=jnp.float32)


def _node_matmuls(x_or_accden, w, a1s, a1d, first):
    bn = 2000
    grid = (N // bn,)
    dout = w.shape[1]
    wspec = [
        pl.BlockSpec(w.shape, lambda i: (0, 0)),
        pl.BlockSpec(a1s.shape, lambda i: (0, 0)),
        pl.BlockSpec(a1d.shape, lambda i: (0, 0)),
    ]
    out_shape = [
        jax.ShapeDtypeStruct((N, dout), jnp.float32),
        jax.ShapeDtypeStruct((N, MLP), jnp.float32),
        jax.ShapeDtypeStruct((N, MLP), jnp.float32),
    ]
    out_specs = [
        pl.BlockSpec((bn, dout), lambda i: (i, 0)),
        pl.BlockSpec((bn, MLP), lambda i: (i, 0)),
        pl.BlockSpec((bn, MLP), lambda i: (i, 0)),
    ]
    if first:
        x = x_or_accden
        return pl.pallas_call(
            _k1_body_first, grid=grid,
            in_specs=[pl.BlockSpec((bn, x.shape[1]), lambda i: (i, 0))] + wspec,
            out_specs=out_specs, out_shape=out_shape,
        )(x, w, a1s, a1d)
    acc, den = x_or_accden
    return pl.pallas_call(
        _k1_body_mid, grid=grid,
        in_specs=[pl.BlockSpec((2, bn, 64), lambda i: (0, i, 0)),
                  pl.BlockSpec((2, bn, 1), lambda i: (0, i, 0))] + wspec,
        out_specs=out_specs, out_shape=out_shape,
    )(acc, den, w, a1s, a1d)


def _k3_body(sg_ref, dg_ref, ea_ref, a1e_ref, b1_ref, a2_ref, b2_ref,
             a3_ref, b3_ref, lg_ref, k_ref):
    i = pl.program_id(0)
    z = sg_ref[...] + dg_ref[...]
    z = z + jnp.dot(ea_ref[...], a1e_ref[...],
                    preferred_element_type=jnp.float32)
    z = jnp.maximum(z + b1_ref[...], 0.0)
    z = jnp.maximum(jnp.dot(z, a2_ref[...],
                            preferred_element_type=jnp.float32) + b2_ref[...],
                    0.0)
    lg = jnp.dot(z, a3_ref[...], preferred_element_type=jnp.float32) \
        + b3_ref[...]
    lg = jnp.where(lg >= 0.0, lg, 0.2 * lg)
    lg_ref[...] = lg
    bmax = jnp.full((1, 16), jnp.max(lg), jnp.float32)

    @pl.when(i == 0)
    def _():
        k_ref[...] = bmax

    @pl.when(i > 0)
    def _():
        k_ref[...] = jnp.maximum(k_ref[...], bmax)


def _edge_mlp(sg, dg, eav, a1e_bd, b1t, a2_bd, b2t, a3_bd, b3t):
    e8 = E // 8
    be = 4000
    grid = (e8 // be,)
    lg, kmax = pl.pallas_call(
        _k3_body, grid=grid,
        in_specs=[
            pl.BlockSpec((be, 128), lambda i: (i, 0)),
            pl.BlockSpec((be, 128), lambda i: (i, 0)),
            pl.BlockSpec((be, 32), lambda i: (i, 0)),
            pl.BlockSpec((32, 128), lambda i: (0, 0)),
            pl.BlockSpec((1, 128), lambda i: (0, 0)),
            pl.BlockSpec((128, 128), lambda i: (0, 0)),
            pl.BlockSpec((1, 128), lambda i: (0, 0)),
            pl.BlockSpec((128, 8), lambda i: (0, 0)),
            pl.BlockSpec((1, 8), lambda i: (0, 0)),
        ],
        out_specs=[pl.BlockSpec((be, 8), lambda i: (i, 0)),
                   pl.BlockSpec((1, 16), lambda i: (0, 0))],
        out_shape=[jax.ShapeDtypeStruct((EP // 8, 8), jnp.float32),
                   jax.ShapeDtypeStruct((1, 16), jnp.float32)],
    )(sg, dg, eav, a1e_bd, b1t, a2_bd, b2t, a3_bd, b3t)
    return lg, kmax


def _epi_body(acc_ref, den_ref, out_ref):
    out_ref[...] = (acc_ref[0] + acc_ref[1]) / (den_ref[0] + den_ref[1] + EPS)


def _epilogue(acc, den):
    bn = 2000
    return pl.pallas_call(
        _epi_body, grid=(N // bn,),
        in_specs=[pl.BlockSpec((2, bn, 64), lambda i: (0, i, 0)),
                  pl.BlockSpec((2, bn, 1), lambda i: (0, i, 0))],
        out_specs=pl.BlockSpec((bn, 64), lambda i: (i, 0)),
        out_shape=jax.ShapeDtypeStruct((N, 64), jnp.float32),
    )(acc, den)


def _k2_sc(src2d, dst2d, s1, d1):
    @functools.partial(
        pl.kernel,
        out_type=(jax.ShapeDtypeStruct((E, MLP), jnp.float32),
                  jax.ShapeDtypeStruct((E, MLP), jnp.float32)),
        mesh=plsc.VectorSubcoreMesh(**_SC_MESH),
        compiler_params=pltpu.CompilerParams(use_tc_tiling_on_sc=False,
                                             needs_layout_passes=False),
        scratch_types=[
            pltpu.VMEM((RPT, 128), jnp.int32),
            pltpu.VMEM((RPT, 128), jnp.int32),
            pltpu.VMEM((128, MLP), jnp.float32),
            pltpu.VMEM((128, MLP), jnp.float32),
            pltpu.VMEM((128, MLP), jnp.float32),
            pltpu.VMEM((128, MLP), jnp.float32),
            pltpu.SemaphoreType.DMA,
            pltpu.SemaphoreType.DMA,
            pltpu.SemaphoreType.DMA,
            pltpu.SemaphoreType.DMA,
        ],
    )
    def body(src_hbm, dst_hbm, s1_hbm, d1_hbm, sg_hbm, dg_hbm,
             idx_s, idx_d, sr0, dr0, sr1, dr1, ss0, sd0, ss1, sd1):
        c = lax.axis_index("c")
        s = lax.axis_index("s")
        w = c * NS + s
        nrows = jnp.clip(ER - w * RPT, 0, RPT)
        pltpu.sync_copy(src_hbm.at[pl.ds(w * RPT, RPT)], idx_s)
        pltpu.sync_copy(dst_hbm.at[pl.ds(w * RPT, RPT)], idx_d)

        pltpu.async_copy(s1_hbm.at[idx_s.at[0]], sr0, ss0)
        pltpu.async_copy(d1_hbm.at[idx_d.at[0]], dr0, sd0)
        pltpu.async_copy(s1_hbm.at[idx_s.at[1]], sr1, ss1)
        pltpu.async_copy(d1_hbm.at[idx_d.at[1]], dr1, sd1)

        def half(j, sr, dr, ss, sd):
            ar = w * RPT + j
            pltpu.make_async_copy(s1_hbm.at[idx_s.at[j]], sr, ss).wait()
            pltpu.make_async_copy(d1_hbm.at[idx_d.at[j]], dr, sd).wait()
            pltpu.sync_copy(sr, sg_hbm.at[pl.ds(ar * 128, 128)])
            pltpu.sync_copy(dr, dg_hbm.at[pl.ds(ar * 128, 128)])

            @pl.when(j + 2 < nrows)
            def _():
                pltpu.async_copy(s1_hbm.at[idx_s.at[j + 2]], sr, ss)
                pltpu.async_copy(d1_hbm.at[idx_d.at[j + 2]], dr, sd)

        def pair(jj, carry):
            half(2 * jj, sr0, dr0, ss0, sd0)
            half(2 * jj + 1, sr1, dr1, ss1, sd1)
            return carry

        lax.fori_loop(0, nrows // 2, pair, 0)

    return body(src2d, dst2d, s1, d1)


def _k5_sc(src2d, dst2d, h, lg2d, kmax):
    @functools.partial(
        pl.kernel,
        out_type=(jax.ShapeDtypeStruct((NC, N, 64), jnp.float32),
                  jax.ShapeDtypeStruct((NC * N,), jnp.float32)),
        mesh=plsc.VectorSubcoreMesh(**_SC_MESH),
        compiler_params=pltpu.CompilerParams(use_tc_tiling_on_sc=False,
                                             needs_layout_passes=False),
        scratch_types=[
            pltpu.VMEM_SHARED((N, 64), jnp.float32),
            pltpu.VMEM_SHARED((N,), jnp.float32),
            pltpu.VMEM((RPT, 128), jnp.int32),
            pltpu.VMEM((RPT, 128), jnp.int32),
            pltpu.VMEM((RPT, 128), jnp.float32),
            pltpu.VMEM((128, 64), jnp.float32),
            pltpu.VMEM((128, 64), jnp.float32),
            pltpu.VMEM((128,), jnp.float32),
            pltpu.VMEM((1, 16), jnp.float32),
            pltpu.SemaphoreType.DMA,
            pltpu.SemaphoreType.DMA,
        ],
    )
    def body(src_hbm, dst_hbm, h_hbm, lg_hbm, k_hbm, acc_out, den_out,
             acc_sh, den_sh, idx_s, idx_d, exb, rows0, rows1, zb1, kbuf,
             sg0, sg1):
        c = lax.axis_index("c")
        s = lax.axis_index("s")
        w = c * NS + s
        nrows = jnp.clip(ER - w * RPT, 0, RPT)

        def zrow(r, carry):
            for q in range(4):
                rows0[r, pl.ds(q * 16, 16)] = jnp.zeros((16,), jnp.float32)
            return carry

        lax.fori_loop(0, 128, zrow, 0)
        for q in range(8):
            zb1[pl.ds(q * 16, 16)] = jnp.zeros((16,), jnp.float32)
        band = (s * 625) // 8 * 8
        for t in range(4):
            pltpu.sync_copy(rows0, acc_sh.at[pl.ds(band + t * 128, 128)])
        pltpu.sync_copy(rows0.at[pl.ds(0, 120)],
                        acc_sh.at[pl.ds(band + 512, 120)])
        for t in range(4):
            pltpu.sync_copy(zb1, den_sh.at[pl.ds(band + t * 128, 128)])
        pltpu.sync_copy(zb1.at[pl.ds(0, 120)],
                        den_sh.at[pl.ds(band + 512, 120)])
        plsc.subcore_barrier()

        pltpu.sync_copy(src_hbm.at[pl.ds(w * RPT, RPT)], idx_s)
        pltpu.sync_copy(dst_hbm.at[pl.ds(w * RPT, RPT)], idx_d)
        pltpu.sync_copy(lg_hbm.at[pl.ds(w * RPT, RPT)], exb)
        pltpu.sync_copy(k_hbm, kbuf)
        kv = kbuf[0, :]

        def erow(j, carry):
            for q in range(8):
                exb[j, pl.ds(q * 16, 16)] = \
                    jnp.exp(exb[j, pl.ds(q * 16, 16)] - kv)
            return carry

        lax.fori_loop(0, nrows, erow, 0)

        pltpu.async_copy(h_hbm.at[idx_s.at[0]], rows0, sg0)
        pltpu.async_copy(h_hbm.at[idx_s.at[1]], rows1, sg1)

        def half(j, rows, sg):
            pltpu.make_async_copy(h_hbm.at[idx_s.at[j]], rows, sg).wait()

            def scale(g, carry2):
                exv = exb[j, pl.ds(g * 16, 16)]
                for d in range(16):
                    bex = exv[jnp.full((16,), d, jnp.int32)]
                    e = g * 16 + d
                    for q in range(4):
                        rows[e, pl.ds(q * 16, 16)] = \
                            rows[e, pl.ds(q * 16, 16)] * bex
                return carry2

            lax.fori_loop(0, 8, scale, 0)
            pltpu.sync_copy(rows, acc_sh.at[idx_d.at[j]], add=True)
            pltpu.sync_copy(exb.at[j], den_sh.at[idx_d.at[j]], add=True)

            @pl.when(j + 2 < nrows)
            def _():
                pltpu.async_copy(h_hbm.at[idx_s.at[j + 2]], rows, sg)

        def pair(jj, carry):
            half(2 * jj, rows0, sg0)
            half(2 * jj + 1, rows1, sg1)
            return carry

        lax.fori_loop(0, nrows // 2, pair, 0)
        plsc.subcore_barrier()

        @pl.when(s == 0)
        def _():
            pltpu.sync_copy(acc_sh, acc_out.at[c])
            pltpu.sync_copy(den_sh, den_out.at[pl.ds(c * N, N)])

    return body(src2d, dst2d, h, lg2d, kmax)


def _block_diag(m, k):
    din, dout = m.shape
    out = jnp.zeros((din * k, dout * k), jnp.float32)
    for i in range(k):
        out = out.at[i * din:(i + 1) * din, i * dout:(i + 1) * dout].set(m)
    return out


def kernel(x, edge_index, edge_attr, params):
    zpad_i = jnp.zeros((EPAD,), jnp.int32)
    src2d = jnp.concatenate(
        [edge_index[0].astype(jnp.int32), zpad_i]).reshape(ERP, 128)
    dst2d = jnp.concatenate(
        [edge_index[1].astype(jnp.int32), zpad_i]).reshape(ERP, 128)
    eav = edge_attr.reshape(E // 8, 8 * D_EDGE)

    acc = den = None
    for li, p in enumerate(params):
        dout = p['W'].shape[1]
        a1 = p['A1']
        a1s, a1d, a1e = a1[:dout], a1[dout:2 * dout], a1[2 * dout:]
        a1e_bd = _block_diag(a1e, 8)
        b1t = jnp.tile(p['b1'], 8)[None, :]
        a2_bd = _block_diag(p['A2'], 8)
        b2t = jnp.tile(p['b2'], 8)[None, :]
        a3_bd = _block_diag(p['A3'], 8)
        b3t = jnp.tile(p['b3'], 8)[None, :]

        if li == 0:
            h, s1, d1 = _node_matmuls(x, p['W'], a1s, a1d, first=True)
        else:
            h, s1, d1 = _node_matmuls((acc, den), p['W'], a1s, a1d,
                                      first=False)

        sg, dg = _k2_sc(src2d, dst2d, s1, d1)
        lg, kmax = _edge_mlp(sg.reshape(E // 8, 128), dg.reshape(E // 8, 128),
                             eav, a1e_bd, b1t, a2_bd, b2t, a3_bd, b3t)
        acc, den_flat = _k5_sc(src2d, dst2d, h, lg.reshape(EP // 128, 128),
                               kmax)
        den = den_flat.reshape(NC, N, 1)

    return _epilogue(acc, den)
